# baseline (speedup 1.0000x reference)
import jax
import jax.numpy as jnp
from jax import lax
from jax.experimental import pallas as pl
from jax.experimental.pallas import tpu as pltpu
from jax.experimental.pallas import tpu_sc as plsc

_SEQLEN = 2048
_EMBED = 768
_BATCH = 4

_NUM_WORKERS = 32
_TOKENS = _BATCH * _SEQLEN
_POSW = _SEQLEN // _NUM_WORKERS
_GROUP = 8
_NGROUPS = _POSW // _GROUP
_LANES = 16


def _emb_body(ids_hbm, wte_hbm, wpe_hbm, out_hbm,
              idx_v, pos0, pos1,
              r00, r01, r02, r03, r10, r11, r12, r13,
              isem, psem0, psem1,
              gsem0, gsem1, ssem0, ssem1):
    wid = lax.axis_index("s") * 2 + lax.axis_index("c")
    pbase = wid * _POSW
    pos = (pos0, pos1)
    rows = ((r00, r01, r02, r03), (r10, r11, r12, r13))
    psems = (psem0, psem1)
    gsems = (gsem0, gsem1)
    ssems = (ssem0, ssem1)

    idx_cps = [pltpu.async_copy(
        ids_hbm.at[b, pl.ds(pbase, _POSW)],
        idx_v.at[pl.ds(b * _POSW, _POSW)], isem) for b in range(_BATCH)]
    for cp in idx_cps:
        cp.wait()

    poscps = [None] * _NGROUPS
    gads = [[None] * _BATCH for _ in range(_NGROUPS)]
    stores = [[None] * _BATCH for _ in range(_NGROUPS)]
    for step in range(_NGROUPS + 1):
        if step < _NGROUPS:
            q = step % 2
            if step >= 2:
                for cp in stores[step - 2]:
                    cp.wait()
            poscps[step] = pltpu.async_copy(
                wpe_hbm.at[pl.ds(pbase + step * _GROUP, _GROUP)],
                pos[q], psems[q])
            for b in range(_BATCH):
                isl = pl.ds(b * _POSW + step * _GROUP, _GROUP)
                gads[step][b] = pltpu.async_copy(
                    wte_hbm.at[idx_v.at[isl]], rows[q][b], gsems[q])
        u = step - 1
        if 0 <= u < _NGROUPS:
            q = u % 2
            poscps[u].wait()
            for b in range(_BATCH):
                gads[u][b].wait()

            @plsc.parallel_loop(0, _GROUP, 1, unroll=2)
            def add_row(r):
                for k in range(_EMBED // _LANES):
                    sl = pl.ds(k * _LANES, _LANES)
                    p = pos[q][r, sl]
                    for b in range(_BATCH):
                        rows[q][b][r, sl] = rows[q][b][r, sl] + p

            for b in range(_BATCH):
                off = pbase + u * _GROUP
                stores[u][b] = pltpu.async_copy(
                    rows[q][b], out_hbm.at[b, pl.ds(off, _GROUP)], ssems[q])
    for u in (_NGROUPS - 2, _NGROUPS - 1):
        for cp in stores[u]:
            cp.wait()


@jax.jit
def kernel(input_ids, token_embeddings, position_embeddings):
    mesh = plsc.VectorSubcoreMesh(core_axis_name="c", subcore_axis_name="s")
    out = pl.kernel(
        _emb_body,
        out_type=jax.ShapeDtypeStruct((_BATCH, _SEQLEN, _EMBED), jnp.float32),
        mesh=mesh,
        scratch_types=(
            [pltpu.VMEM((_BATCH * _POSW,), jnp.int32)]
            + [pltpu.VMEM((_GROUP, _EMBED), jnp.float32)] * 2
            + [pltpu.VMEM((_GROUP, _EMBED), jnp.float32)] * (2 * _BATCH)
            + [pltpu.SemaphoreType.DMA] * 7
        ),
    )(input_ids, token_embeddings, position_embeddings)
    return out


# speedup vs baseline: 1.0476x; 1.0476x over previous
import jax
import jax.numpy as jnp
from jax import lax
from jax.experimental import pallas as pl
from jax.experimental.pallas import tpu as pltpu
from jax.experimental.pallas import tpu_sc as plsc

_SEQLEN = 2048
_EMBED = 768
_BATCH = 4

_NUM_WORKERS = 32
_TOKENS = _BATCH * _SEQLEN
_POSW = _SEQLEN // _NUM_WORKERS
_GROUP = 16
_NGROUPS = _POSW // _GROUP
_LANES = 16


def _emb_body(ids_hbm, wte_hbm, wpe_hbm, out_hbm,
              idx_v, pos0, pos1,
              r00, r01, r02, r03, r10, r11, r12, r13,
              isem, psem0, psem1,
              gsem0, gsem1, ssem0, ssem1):
    wid = lax.axis_index("s") * 2 + lax.axis_index("c")
    pbase = wid * _POSW
    pos = (pos0, pos1)
    rows = ((r00, r01, r02, r03), (r10, r11, r12, r13))
    psems = (psem0, psem1)
    gsems = (gsem0, gsem1)
    ssems = (ssem0, ssem1)

    idx_cps = [pltpu.async_copy(
        ids_hbm.at[b, pl.ds(pbase, _POSW)],
        idx_v.at[pl.ds(b * _POSW, _POSW)], isem) for b in range(_BATCH)]
    for cp in idx_cps:
        cp.wait()

    poscps = [None] * _NGROUPS
    gads = [[None] * _BATCH for _ in range(_NGROUPS)]
    stores = [[None] * _BATCH for _ in range(_NGROUPS)]
    for step in range(_NGROUPS + 1):
        if step < _NGROUPS:
            q = step % 2
            if step >= 2:
                for cp in stores[step - 2]:
                    cp.wait()
            poscps[step] = pltpu.async_copy(
                wpe_hbm.at[pl.ds(pbase + step * _GROUP, _GROUP)],
                pos[q], psems[q])
            for b in range(_BATCH):
                isl = pl.ds(b * _POSW + step * _GROUP, _GROUP)
                gads[step][b] = pltpu.async_copy(
                    wte_hbm.at[idx_v.at[isl]], rows[q][b], gsems[q])
        u = step - 1
        if 0 <= u < _NGROUPS:
            q = u % 2
            poscps[u].wait()
            for b in range(_BATCH):
                gads[u][b].wait()

            @plsc.parallel_loop(0, _GROUP, 1, unroll=2)
            def add_row(r):
                for k in range(_EMBED // _LANES):
                    sl = pl.ds(k * _LANES, _LANES)
                    p = pos[q][r, sl]
                    for b in range(_BATCH):
                        rows[q][b][r, sl] = rows[q][b][r, sl] + p

            for b in range(_BATCH):
                off = pbase + u * _GROUP
                stores[u][b] = pltpu.async_copy(
                    rows[q][b], out_hbm.at[b, pl.ds(off, _GROUP)], ssems[q])
    for u in (_NGROUPS - 2, _NGROUPS - 1):
        for cp in stores[u]:
            cp.wait()


@jax.jit
def kernel(input_ids, token_embeddings, position_embeddings):
    mesh = plsc.VectorSubcoreMesh(core_axis_name="c", subcore_axis_name="s")
    out = pl.kernel(
        _emb_body,
        out_type=jax.ShapeDtypeStruct((_BATCH, _SEQLEN, _EMBED), jnp.float32),
        mesh=mesh,
        scratch_types=(
            [pltpu.VMEM((_BATCH * _POSW,), jnp.int32)]
            + [pltpu.VMEM((_GROUP, _EMBED), jnp.float32)] * 2
            + [pltpu.VMEM((_GROUP, _EMBED), jnp.float32)] * (2 * _BATCH)
            + [pltpu.SemaphoreType.DMA] * 7
        ),
    )(input_ids, token_embeddings, position_embeddings)
    return out
